# Initial kernel scaffold; baseline (speedup 1.0000x reference)
import functools

import jax
import jax.numpy as jnp
from jax import lax
from jax.experimental import pallas as pl
from jax.experimental.pallas import tpu as pltpu
from jax.experimental.pallas import tpu_sc as plsc

N_WORKERS = 32
LANES = 16
IDX_CHUNK = 128


def _face_normals_sc(n_verts, n_faces):
  fpw = n_faces // N_WORKERS
  idx_per_w = fpw * 3
  n_chunks = (idx_per_w + IDX_CHUNK - 1) // IDX_CHUNK
  idx_pad = n_chunks * IDX_CHUNK
  n_iters = (fpw + LANES - 1) // LANES
  out_pad = n_iters * LANES * 3

  mesh = plsc.VectorSubcoreMesh(core_axis_name="c", subcore_axis_name="s")

  @functools.partial(
      pl.kernel,
      mesh=mesh,
      out_type=jax.ShapeDtypeStruct((N_WORKERS, idx_per_w), jnp.float32),
      scratch_types=[
          pltpu.VMEM((idx_pad,), jnp.int32),
          pltpu.VMEM((idx_pad, 4), jnp.float32),
          pltpu.VMEM((out_pad,), jnp.float32),
          pltpu.SemaphoreType.DMA,
      ],
  )
  def k(vert_hbm, faces_hbm, out_hbm, idx_v, rows_v, out_v, sem):
    wid = lax.axis_index("s") * 2 + lax.axis_index("c")

    pltpu.sync_copy(faces_hbm.at[wid], idx_v)

    def fire(c, _):
      src = vert_hbm.at[idx_v.at[pl.ds(c * IDX_CHUNK, IDX_CHUNK)]]
      pltpu.async_copy(src, rows_v.at[pl.ds(c * IDX_CHUNK, IDX_CHUNK)], sem)
      return _
    lax.fori_loop(0, n_chunks, fire, None)

    def drain(c, _):
      src = vert_hbm.at[idx_v.at[pl.ds(c * IDX_CHUNK, IDX_CHUNK)]]
      pltpu.make_async_copy(
          src, rows_v.at[pl.ds(c * IDX_CHUNK, IDX_CHUNK)], sem).wait()
      return _
    lax.fori_loop(0, n_chunks, drain, None)

    iota = lax.iota(jnp.int32, LANES)
    c0 = jnp.zeros((LANES,), jnp.int32)
    c1 = c0 + 1
    c2 = c0 + 2

    def body(i, _):
      f3 = (i * LANES + iota) * 3
      v0x = plsc.load_gather(rows_v, [f3, c0])
      v0y = plsc.load_gather(rows_v, [f3, c1])
      v0z = plsc.load_gather(rows_v, [f3, c2])
      v1x = plsc.load_gather(rows_v, [f3 + 1, c0])
      v1y = plsc.load_gather(rows_v, [f3 + 1, c1])
      v1z = plsc.load_gather(rows_v, [f3 + 1, c2])
      v2x = plsc.load_gather(rows_v, [f3 + 2, c0])
      v2y = plsc.load_gather(rows_v, [f3 + 2, c1])
      v2z = plsc.load_gather(rows_v, [f3 + 2, c2])
      e1x, e1y, e1z = v1x - v0x, v1y - v0y, v1z - v0z
      e2x, e2y, e2z = v2x - v0x, v2y - v0y, v2z - v0z
      nx = e1y * e2z - e1z * e2y
      ny = e1z * e2x - e1x * e2z
      nz = e1x * e2y - e1y * e2x
      s = nx * nx + ny * ny + nz * nz
      bi = jnp.int32(0x5F3759DF) - lax.shift_right_logical(
          plsc.bitcast(s, jnp.int32), 1)
      y = plsc.bitcast(bi, jnp.float32)
      half_s = 0.5 * s
      y = y * (1.5 - half_s * y * y)
      y = y * (1.5 - half_s * y * y)
      y = y * (1.5 - half_s * y * y)
      norm = s * y
      r = 1.0 / jnp.maximum(norm, 1e-6)
      plsc.store_scatter(out_v, [f3], nx * r)
      plsc.store_scatter(out_v, [f3 + 1], ny * r)
      plsc.store_scatter(out_v, [f3 + 2], nz * r)
      return _
    lax.fori_loop(0, n_iters, body, None)

    pltpu.sync_copy(out_v.at[pl.ds(0, idx_per_w)], out_hbm.at[wid])

  return k, fpw, idx_per_w, idx_pad


def kernel(vertices, faces):
  n_verts = vertices.shape[1]
  n_faces = faces.shape[0]
  k, _, idx_per_w, idx_pad = _face_normals_sc(n_verts, n_faces)

  v = jnp.concatenate(
      [vertices[0], jnp.zeros((n_verts, 1), jnp.float32)], axis=1)
  flat = faces.reshape(-1)
  flat = jnp.pad(flat, (0, N_WORKERS * idx_pad - flat.shape[0]))
  faces_w = flat.reshape(N_WORKERS, idx_pad)

  out = k(v, faces_w)
  return out.reshape(n_faces, 3)


# speedup vs baseline: 7.6511x; 7.6511x over previous
import functools

import jax
import jax.numpy as jnp
from jax import lax
from jax.experimental import pallas as pl
from jax.experimental.pallas import tpu as pltpu
from jax.experimental.pallas import tpu_sc as plsc

N_WORKERS = 32
LANES = 16
IDX_CHUNK = 128
BLOCK_FACES = 640
ROW_W = 8


def _face_normals_sc(n_verts, n_faces):
  fpw = n_faces // N_WORKERS
  n_blocks = (fpw + BLOCK_FACES - 1) // BLOCK_FACES
  fpw_pad = n_blocks * BLOCK_FACES
  idx_per_w = fpw * 3
  idx_pad = fpw_pad * 3
  blk_idx = BLOCK_FACES * 3
  n_chunks = blk_idx // IDX_CHUNK
  n_iters = BLOCK_FACES // LANES

  mesh = plsc.VectorSubcoreMesh(core_axis_name="c", subcore_axis_name="s")

  @functools.partial(
      pl.kernel,
      mesh=mesh,
      compiler_params=pltpu.CompilerParams(
          needs_layout_passes=False, use_tc_tiling_on_sc=False),
      out_type=jax.ShapeDtypeStruct((N_WORKERS, idx_per_w), jnp.float32),
      scratch_types=[
          pltpu.VMEM((idx_pad,), jnp.int32),
          pltpu.VMEM((blk_idx, ROW_W), jnp.float32),
          pltpu.VMEM((blk_idx, ROW_W), jnp.float32),
          pltpu.VMEM((idx_pad,), jnp.float32),
          pltpu.SemaphoreType.DMA,
          pltpu.SemaphoreType.DMA,
      ],
  )
  def k(vert_hbm, faces_hbm, out_hbm, idx_v, rows0, rows1, out_v, sem0, sem1):
    wid = lax.axis_index("s") * 2 + lax.axis_index("c")
    rows = (rows0, rows1)
    sems = (sem0, sem1)

    pltpu.sync_copy(faces_hbm.at[wid], idx_v)

    def fire_block(b):
      buf, sem = rows[b % 2], sems[b % 2]
      def fire(c, _):
        src = vert_hbm.at[idx_v.at[pl.ds(b * blk_idx + c * IDX_CHUNK,
                                         IDX_CHUNK)]]
        pltpu.async_copy(src, buf.at[pl.ds(c * IDX_CHUNK, IDX_CHUNK)], sem)
        return _
      lax.fori_loop(0, n_chunks, fire, None)

    def drain_block(b):
      buf, sem = rows[b % 2], sems[b % 2]
      def drain(c, _):
        src = vert_hbm.at[idx_v.at[pl.ds(b * blk_idx + c * IDX_CHUNK,
                                         IDX_CHUNK)]]
        pltpu.make_async_copy(
            src, buf.at[pl.ds(c * IDX_CHUNK, IDX_CHUNK)], sem).wait()
        return _
      lax.fori_loop(0, n_chunks, drain, None)

    iota = lax.iota(jnp.int32, LANES)
    c0 = jnp.zeros((LANES,), jnp.int32)
    c1 = c0 + 1
    c2 = c0 + 2

    def compute_block(b):
      buf = rows[b % 2]
      out_base = b * blk_idx

      def body(i, _):
        lf3 = (i * LANES + iota) * 3
        v0x = plsc.load_gather(buf, [lf3, c0])
        v0y = plsc.load_gather(buf, [lf3, c1])
        v0z = plsc.load_gather(buf, [lf3, c2])
        v1x = plsc.load_gather(buf, [lf3 + 1, c0])
        v1y = plsc.load_gather(buf, [lf3 + 1, c1])
        v1z = plsc.load_gather(buf, [lf3 + 1, c2])
        v2x = plsc.load_gather(buf, [lf3 + 2, c0])
        v2y = plsc.load_gather(buf, [lf3 + 2, c1])
        v2z = plsc.load_gather(buf, [lf3 + 2, c2])
        e1x, e1y, e1z = v1x - v0x, v1y - v0y, v1z - v0z
        e2x, e2y, e2z = v2x - v0x, v2y - v0y, v2z - v0z
        nx = e1y * e2z - e1z * e2y
        ny = e1z * e2x - e1x * e2z
        nz = e1x * e2y - e1y * e2x
        s = nx * nx + ny * ny + nz * nz
        bi = jnp.int32(0x5F3759DF) - lax.shift_right_logical(
            plsc.bitcast(s, jnp.int32), 1)
        y = plsc.bitcast(bi, jnp.float32)
        half_s = 0.5 * s
        y = y * (1.5 - half_s * y * y)
        y = y * (1.5 - half_s * y * y)
        y = y * (1.5 - half_s * y * y)
        norm = s * y
        r = 1.0 / jnp.maximum(norm, 1e-6)
        g3 = out_base + lf3
        plsc.store_scatter(out_v, [g3], nx * r)
        plsc.store_scatter(out_v, [g3 + 1], ny * r)
        plsc.store_scatter(out_v, [g3 + 2], nz * r)
        return _
      lax.fori_loop(0, n_iters, body, None)

    fire_block(0)
    for b in range(n_blocks):
      if b + 1 < n_blocks:
        fire_block(b + 1)
      drain_block(b)
      compute_block(b)

    pltpu.sync_copy(out_v.at[pl.ds(0, idx_per_w)], out_hbm.at[wid])

  return k, idx_per_w, idx_pad


def kernel(vertices, faces):
  n_verts = vertices.shape[1]
  n_faces = faces.shape[0]
  k, idx_per_w, idx_pad = _face_normals_sc(n_verts, n_faces)

  v = jnp.pad(vertices[0], ((0, 0), (0, ROW_W - 3)))
  flat = faces.reshape(N_WORKERS, idx_per_w)
  faces_w = jnp.pad(flat, ((0, 0), (0, idx_pad - idx_per_w)))

  out = k(v, faces_w)
  return out.reshape(n_faces, 3)


# speedup vs baseline: 7.9785x; 1.0428x over previous
import functools

import jax
import jax.numpy as jnp
from jax import lax
from jax.experimental import pallas as pl
from jax.experimental.pallas import tpu as pltpu
from jax.experimental.pallas import tpu_sc as plsc

N_WORKERS = 32
LANES = 16
IDX_CHUNK = 128
BLOCK_FACES = 640
ROW_W = 8


def _face_normals_sc(n_verts, n_faces):
  fpw0 = (n_faces // N_WORKERS) // 8 * 8
  fpw_last = n_faces - (N_WORKERS - 1) * fpw0
  n_blocks = -(-fpw_last // BLOCK_FACES)
  fpw_pad = n_blocks * BLOCK_FACES
  idx0 = fpw0 * 3
  idx_last = fpw_last * 3
  idx_pad = fpw_pad * 3
  blk_idx = BLOCK_FACES * 3
  n_chunks = blk_idx // IDX_CHUNK
  n_iters = BLOCK_FACES // LANES
  flat_pad = (N_WORKERS - 1) * idx0 + idx_pad

  mesh = plsc.VectorSubcoreMesh(core_axis_name="c", subcore_axis_name="s")

  @functools.partial(
      pl.kernel,
      mesh=mesh,
      compiler_params=pltpu.CompilerParams(
          needs_layout_passes=False, use_tc_tiling_on_sc=False),
      out_type=jax.ShapeDtypeStruct((n_faces * 3,), jnp.float32),
      scratch_types=[
          pltpu.VMEM((idx_pad,), jnp.int32),
          pltpu.VMEM((blk_idx, ROW_W), jnp.float32),
          pltpu.VMEM((blk_idx, ROW_W), jnp.float32),
          pltpu.VMEM((idx_pad,), jnp.float32),
          pltpu.SemaphoreType.DMA,
          pltpu.SemaphoreType.DMA,
      ],
  )
  def k(vert_hbm, faces_hbm, out_hbm,
        idx_v, rows0, rows1, out_v, sem0, sem1):
    sid = lax.axis_index("s")
    cid = lax.axis_index("c")
    wid = sid * 2 + cid
    base = wid * idx0
    rows = (rows0, rows1)
    sems = (sem0, sem1)

    pltpu.sync_copy(faces_hbm.at[pl.ds(base, idx_pad)], idx_v)

    def fire_block(b):
      buf, sem = rows[b % 2], sems[b % 2]
      def fire(c, _):
        src = vert_hbm.at[idx_v.at[pl.ds(b * blk_idx + c * IDX_CHUNK,
                                         IDX_CHUNK)]]
        pltpu.async_copy(src, buf.at[pl.ds(c * IDX_CHUNK, IDX_CHUNK)], sem)
        return _
      lax.fori_loop(0, n_chunks, fire, None)

    def drain_block(b):
      buf, sem = rows[b % 2], sems[b % 2]
      def drain(c, _):
        src = vert_hbm.at[idx_v.at[pl.ds(b * blk_idx + c * IDX_CHUNK,
                                         IDX_CHUNK)]]
        pltpu.make_async_copy(
            src, buf.at[pl.ds(c * IDX_CHUNK, IDX_CHUNK)], sem).wait()
        return _
      lax.fori_loop(0, n_chunks, drain, None)

    iota = lax.iota(jnp.int32, LANES)
    c0 = jnp.zeros((LANES,), jnp.int32)
    c1 = c0 + 1
    c2 = c0 + 2

    def compute_block(b):
      buf = rows[b % 2]
      out_base = b * blk_idx

      def body(i, _):
        lf3 = (i * LANES + iota) * 3
        v0x = plsc.load_gather(buf, [lf3, c0])
        v0y = plsc.load_gather(buf, [lf3, c1])
        v0z = plsc.load_gather(buf, [lf3, c2])
        v1x = plsc.load_gather(buf, [lf3 + 1, c0])
        v1y = plsc.load_gather(buf, [lf3 + 1, c1])
        v1z = plsc.load_gather(buf, [lf3 + 1, c2])
        v2x = plsc.load_gather(buf, [lf3 + 2, c0])
        v2y = plsc.load_gather(buf, [lf3 + 2, c1])
        v2z = plsc.load_gather(buf, [lf3 + 2, c2])
        e1x, e1y, e1z = v1x - v0x, v1y - v0y, v1z - v0z
        e2x, e2y, e2z = v2x - v0x, v2y - v0y, v2z - v0z
        nx = e1y * e2z - e1z * e2y
        ny = e1z * e2x - e1x * e2z
        nz = e1x * e2y - e1y * e2x
        s = nx * nx + ny * ny + nz * nz
        bi = jnp.int32(0x5F3759DF) - lax.shift_right_logical(
            plsc.bitcast(s, jnp.int32), 1)
        y = plsc.bitcast(bi, jnp.float32)
        half_s = 0.5 * s
        y = y * (1.5 - half_s * y * y)
        y = y * (1.5 - half_s * y * y)
        y = y * (1.5 - half_s * y * y)
        norm = s * y
        r = 1.0 / jnp.maximum(norm, 1e-6)
        g3 = out_base + lf3
        plsc.store_scatter(out_v, [g3], nx * r)
        plsc.store_scatter(out_v, [g3 + 1], ny * r)
        plsc.store_scatter(out_v, [g3 + 2], nz * r)
        return _
      lax.fori_loop(0, n_iters, body, None)

    fire_block(0)
    for b in range(n_blocks):
      if b + 1 < n_blocks:
        fire_block(b + 1)
      drain_block(b)
      compute_block(b)

    pltpu.sync_copy(out_v.at[pl.ds(0, idx0)], out_hbm.at[pl.ds(base, idx0)])
    @pl.when(wid == N_WORKERS - 1)
    def _():
      pltpu.sync_copy(out_v.at[pl.ds(idx0, idx_last - idx0)],
                      out_hbm.at[pl.ds(base + idx0, idx_last - idx0)])

  return k, flat_pad


def kernel(vertices, faces):
  n_verts = vertices.shape[1]
  n_faces = faces.shape[0]
  k, flat_pad = _face_normals_sc(n_verts, n_faces)

  v = jnp.pad(vertices[0], ((0, 0), (0, ROW_W - 3)))
  flat = faces.reshape(-1)
  flat = jnp.pad(flat, (0, flat_pad - flat.shape[0]))

  out = k(v, flat)
  return out.reshape(n_faces, 3)
